# baseline (speedup 1.0000x reference)
import jax
import jax.numpy as jnp
from jax import lax
from jax.experimental import pallas as pl
from jax.experimental.pallas import tpu as pltpu
from jax.experimental.pallas import tpu_sc as plsc

N = 10000
E = 160000
IN = 256
H = 256
OUT = 128
D = H // 2

NC = 2
NS = 16
K = 64
EPT = 10240
NB = EPT // K
EPAD = EPT * NS
NROW = 10240
ZB = NROW // (K * NS)
CPT = NROW // NS
NBUF = 5

BN = 1000
GRID = N // BN

_SDS = jax.ShapeDtypeStruct
_MESH = plsc.VectorSubcoreMesh(core_axis_name="c", subcore_axis_name="s")


CH = 10
NCH = NB // CH


def _agg_body(z0_hbm, z1_hbm, slab_hbm, a0_hbm, a1_hbm,
              acc, idx, rows0, rows1, rows2, rows3, rows4,
              semi, sg0, sg1, sg2, sg3, sg4):
    c = lax.axis_index("c")
    s = lax.axis_index("s")
    rows = (rows0, rows1, rows2, rows3, rows4)
    sg = (sg0, sg1, sg2, sg3, sg4)
    zrow = jnp.zeros((16,), jnp.float32)

    def fill_zero(i, _):
        for t in range(D // 16):
            rows0[i, pl.ds(t * 16, 16)] = zrow
        return 0
    lax.fori_loop(0, K, fill_zero, 0)

    def zero_blk(k, _):
        pltpu.sync_copy(rows0, acc.at[pl.ds((s * ZB + k) * K, K)])
        return 0
    lax.fori_loop(0, ZB, zero_blk, 0)

    pltpu.sync_copy(slab_hbm.at[c, s, pl.ds(0, CH)], idx.at[0])
    plsc.subcore_barrier()

    def pipeline(z_hbm):
        def gather(b, p, jj):
            pltpu.async_copy(z_hbm.at[idx.at[p, jj, 0]], rows[b], sg[b])

        def chunk(ci, _):
            p = ci % 2
            pn = (ci + 1) % 2

            @pl.when(ci == 0)
            def _():
                for b in range(NBUF - 1):
                    gather(b, p, b)

            @pl.when(ci + 1 < NCH)
            def _():
                pltpu.async_copy(
                    slab_hbm.at[c, s, pl.ds((ci + 1) * CH, CH)],
                    idx.at[pn], semi)

            for jj in range(CH):
                b = jj % NBUF
                pltpu.make_async_copy(z_hbm.at[idx.at[p, jj, 0]],
                                      rows[b], sg[b]).wait()
                pltpu.sync_copy(rows[b], acc.at[idx.at[p, jj, 1]], add=True)
                nj = jj + NBUF - 1
                if nj < CH:
                    gather(nj % NBUF, p, nj)
                else:
                    if jj == CH - (NBUF - 1):
                        @pl.when(ci + 1 < NCH)
                        def _():
                            pltpu.make_async_copy(
                                slab_hbm.at[c, s, pl.ds(0, CH)],
                                idx.at[pn], semi).wait()

                    @pl.when(ci + 1 < NCH)
                    def _(nj=nj, pn=pn):
                        gather(nj % NBUF, pn, nj - CH)
            return 0
        lax.fori_loop(0, NCH, chunk, 0)

    @pl.when(c == 0)
    def _():
        pipeline(z0_hbm)

    @pl.when(c == 1)
    def _():
        pipeline(z1_hbm)

    plsc.subcore_barrier()

    @pl.when(c == 0)
    def _():
        pltpu.sync_copy(acc.at[pl.ds(s * CPT, CPT)],
                        a0_hbm.at[pl.ds(s * CPT, CPT)])

    @pl.when(c == 1)
    def _():
        pltpu.sync_copy(acc.at[pl.ds(s * CPT, CPT)],
                        a1_hbm.at[pl.ds(s * CPT, CPT)])


_agg = pl.kernel(
    _agg_body,
    out_type=[_SDS((NROW, D), jnp.float32), _SDS((NROW, D), jnp.float32)],
    mesh=_MESH,
    scratch_types=[
        pltpu.VMEM_SHARED((NROW, D), jnp.float32),
        pltpu.VMEM((2, CH, 2, K), jnp.int32),
        pltpu.VMEM((K, D), jnp.float32),
        pltpu.VMEM((K, D), jnp.float32),
        pltpu.VMEM((K, D), jnp.float32),
        pltpu.VMEM((K, D), jnp.float32),
        pltpu.VMEM((K, D), jnp.float32),
        pltpu.SemaphoreType.DMA,
        pltpu.SemaphoreType.DMA,
        pltpu.SemaphoreType.DMA,
        pltpu.SemaphoreType.DMA,
        pltpu.SemaphoreType.DMA,
        pltpu.SemaphoreType.DMA,
    ],
)


def _lp_body(x_ref, w_ref, p_ref):
    p_ref[...] = jnp.dot(x_ref[...], w_ref[...],
                         preferred_element_type=jnp.float32)


_lp = pl.pallas_call(
    _lp_body,
    grid=(GRID,),
    in_specs=[
        pl.BlockSpec((BN, IN), lambda i: (i, 0)),
        pl.BlockSpec((IN, H), lambda i: (0, 0)),
    ],
    out_specs=pl.BlockSpec((BN, H), lambda i: (i, 0)),
    out_shape=_SDS((N, H), jnp.float32),
)


def _l1_body(dego_ref, degi_ref, p_ref, z0_ref, z1_ref,
             ns_ref, nd_ref):
    nsrc = lax.rsqrt(jnp.maximum(dego_ref[..., 0:1], 1.0))
    ndst = lax.rsqrt(jnp.maximum(degi_ref[..., 0:1], 1.0))
    ns_ref[...] = nsrc
    nd_ref[...] = ndst
    z = p_ref[...] * nsrc
    z0_ref[...] = z[:, :D]
    z1_ref[...] = z[:, D:]


_l1 = pl.pallas_call(
    _l1_body,
    grid=(GRID,),
    in_specs=[
        pl.BlockSpec((BN, D), lambda i: (i, 0)),
        pl.BlockSpec((BN, D), lambda i: (i, 0)),
        pl.BlockSpec((BN, H), lambda i: (i, 0)),
    ],
    out_specs=[
        pl.BlockSpec((BN, D), lambda i: (i, 0)),
        pl.BlockSpec((BN, D), lambda i: (i, 0)),
        pl.BlockSpec((BN, 1), lambda i: (i, 0)),
        pl.BlockSpec((BN, 1), lambda i: (i, 0)),
    ],
    out_shape=[
        _SDS((NROW, D), jnp.float32),
        _SDS((NROW, D), jnp.float32),
        _SDS((N, 1), jnp.float32),
        _SDS((N, 1), jnp.float32),
    ],
)


def _make_mid(hout):
    dh = hout // 2
    wpad = D - dh

    def body(a0_ref, a1_ref, nd_ref, ns_ref, b_ref, w_ref, z0_ref, z1_ref):
        h = jnp.concatenate([a0_ref[...], a1_ref[...]], axis=1)
        h = jnp.maximum(h * nd_ref[...] + b_ref[...], 0.0)
        z = jnp.dot(h * ns_ref[...], w_ref[...],
                    preferred_element_type=jnp.float32)
        if wpad:
            zp = jnp.zeros((z.shape[0], wpad), jnp.float32)
            z0_ref[...] = jnp.concatenate([z[:, :dh], zp], axis=1)
            z1_ref[...] = jnp.concatenate([z[:, dh:], zp], axis=1)
        else:
            z0_ref[...] = z[:, :dh]
            z1_ref[...] = z[:, dh:]

    return pl.pallas_call(
        body,
        grid=(GRID,),
        in_specs=[
            pl.BlockSpec((BN, D), lambda i: (i, 0)),
            pl.BlockSpec((BN, D), lambda i: (i, 0)),
            pl.BlockSpec((BN, 1), lambda i: (i, 0)),
            pl.BlockSpec((BN, 1), lambda i: (i, 0)),
            pl.BlockSpec((1, H), lambda i: (0, 0)),
            pl.BlockSpec((H, hout), lambda i: (0, 0)),
        ],
        out_specs=[
            pl.BlockSpec((BN, D), lambda i: (i, 0)),
            pl.BlockSpec((BN, D), lambda i: (i, 0)),
        ],
        out_shape=[
            _SDS((NROW, D), jnp.float32),
            _SDS((NROW, D), jnp.float32),
        ],
    )


_mid256 = _make_mid(H)
_mid128 = _make_mid(OUT)


def _fin_body(a0_ref, a1_ref, nd_ref, b_ref, o_ref):
    i = pl.program_id(0)
    h = jnp.concatenate([a0_ref[..., :OUT // 2], a1_ref[..., :OUT // 2]],
                        axis=1) * nd_ref[...]
    p = jnp.sum(h, axis=0, keepdims=True)

    @pl.when(i == 0)
    def _():
        o_ref[...] = jnp.zeros_like(o_ref)

    o_ref[...] += p

    @pl.when(i == GRID - 1)
    def _():
        o_ref[...] = o_ref[...] * (1.0 / N) + b_ref[...]


_fin = pl.pallas_call(
    _fin_body,
    grid=(GRID,),
    in_specs=[
        pl.BlockSpec((BN, D), lambda i: (i, 0)),
        pl.BlockSpec((BN, D), lambda i: (i, 0)),
        pl.BlockSpec((BN, 1), lambda i: (i, 0)),
        pl.BlockSpec((1, OUT), lambda i: (0, 0)),
    ],
    out_specs=pl.BlockSpec((1, OUT), lambda i: (0, 0)),
    out_shape=_SDS((1, OUT), jnp.float32),
)


def kernel(x, edge_index, W1, b1, W2, b2, W3, b3, W4, b4):
    src = edge_index[0]
    dst = edge_index[1]
    npad = EPAD - E
    fill = jnp.int32(N) + (jnp.arange(npad, dtype=jnp.int32) % 16)
    srcp = jnp.concatenate([src, fill]).reshape(NS, NB, K)
    dstp = jnp.concatenate([dst, fill]).reshape(NS, NB, K)
    lslab = jnp.stack([srcp, dstp], axis=2)
    lslab = jnp.stack([lslab, lslab])
    dslab = jnp.stack([jnp.stack([srcp, srcp], axis=2),
                       jnp.stack([dstp, dstp], axis=2)])

    ones = jnp.ones((NROW, D), jnp.float32)
    p1 = _lp(x, W1)
    dego, degi = _agg(ones, ones, dslab)
    z0, z1, nsrc, ndst = _l1(dego, degi, p1)
    a0, a1 = _agg(z0, z1, lslab)
    z0, z1 = _mid256(a0, a1, ndst, nsrc, b1.reshape(1, H), W2)
    a0, a1 = _agg(z0, z1, lslab)
    z0, z1 = _mid256(a0, a1, ndst, nsrc, b2.reshape(1, H), W3)
    a0, a1 = _agg(z0, z1, lslab)
    y0, y1 = _mid128(a0, a1, ndst, nsrc, b3.reshape(1, H), W4)
    a0, a1 = _agg(y0, y1, lslab)
    out = _fin(a0, a1, ndst, b4.reshape(1, OUT))
    return out.reshape(OUT)


# speedup vs baseline: 1.0155x; 1.0155x over previous
import jax
import jax.numpy as jnp
from jax import lax
from jax.experimental import pallas as pl
from jax.experimental.pallas import tpu as pltpu
from jax.experimental.pallas import tpu_sc as plsc

N = 10000
E = 160000
IN = 256
H = 256
OUT = 128
D = H // 2

NC = 2
NS = 16
K = 64
EPT = 10240
NB = EPT // K
EPAD = EPT * NS
NROW = 10240
ZB = NROW // (K * NS)
CPT = NROW // NS
NBUF = 5

BN = 2000
GRID = N // BN

_SDS = jax.ShapeDtypeStruct
_MESH = plsc.VectorSubcoreMesh(core_axis_name="c", subcore_axis_name="s")


CH = 10
NCH = NB // CH


def _agg_body(z0_hbm, z1_hbm, slab_hbm, a0_hbm, a1_hbm,
              acc, idx, rows0, rows1, rows2, rows3, rows4,
              semi, sg0, sg1, sg2, sg3, sg4):
    c = lax.axis_index("c")
    s = lax.axis_index("s")
    rows = (rows0, rows1, rows2, rows3, rows4)
    sg = (sg0, sg1, sg2, sg3, sg4)
    zrow = jnp.zeros((16,), jnp.float32)

    def fill_zero(i, _):
        for t in range(D // 16):
            rows0[i, pl.ds(t * 16, 16)] = zrow
        return 0
    lax.fori_loop(0, K, fill_zero, 0)

    def zero_blk(k, _):
        pltpu.sync_copy(rows0, acc.at[pl.ds((s * ZB + k) * K, K)])
        return 0
    lax.fori_loop(0, ZB, zero_blk, 0)

    pltpu.sync_copy(slab_hbm.at[c, s, pl.ds(0, CH)], idx.at[0])
    plsc.subcore_barrier()

    def pipeline(z_hbm):
        def gather(b, p, jj):
            pltpu.async_copy(z_hbm.at[idx.at[p, jj, 0]], rows[b], sg[b])

        def chunk(ci, _):
            p = ci % 2
            pn = (ci + 1) % 2

            @pl.when(ci == 0)
            def _():
                for b in range(NBUF - 1):
                    gather(b, p, b)

            @pl.when(ci + 1 < NCH)
            def _():
                pltpu.async_copy(
                    slab_hbm.at[c, s, pl.ds((ci + 1) * CH, CH)],
                    idx.at[pn], semi)

            for jj in range(CH):
                b = jj % NBUF
                pltpu.make_async_copy(z_hbm.at[idx.at[p, jj, 0]],
                                      rows[b], sg[b]).wait()
                pltpu.sync_copy(rows[b], acc.at[idx.at[p, jj, 1]], add=True)
                nj = jj + NBUF - 1
                if nj < CH:
                    gather(nj % NBUF, p, nj)
                else:
                    if jj == CH - (NBUF - 1):
                        @pl.when(ci + 1 < NCH)
                        def _():
                            pltpu.make_async_copy(
                                slab_hbm.at[c, s, pl.ds(0, CH)],
                                idx.at[pn], semi).wait()

                    @pl.when(ci + 1 < NCH)
                    def _(nj=nj, pn=pn):
                        gather(nj % NBUF, pn, nj - CH)
            return 0
        lax.fori_loop(0, NCH, chunk, 0)

    @pl.when(c == 0)
    def _():
        pipeline(z0_hbm)

    @pl.when(c == 1)
    def _():
        pipeline(z1_hbm)

    plsc.subcore_barrier()

    @pl.when(c == 0)
    def _():
        pltpu.sync_copy(acc.at[pl.ds(s * CPT, CPT)],
                        a0_hbm.at[pl.ds(s * CPT, CPT)])

    @pl.when(c == 1)
    def _():
        pltpu.sync_copy(acc.at[pl.ds(s * CPT, CPT)],
                        a1_hbm.at[pl.ds(s * CPT, CPT)])


_agg = pl.kernel(
    _agg_body,
    out_type=[_SDS((NROW, D), jnp.float32), _SDS((NROW, D), jnp.float32)],
    mesh=_MESH,
    scratch_types=[
        pltpu.VMEM_SHARED((NROW, D), jnp.float32),
        pltpu.VMEM((2, CH, 2, K), jnp.int32),
        pltpu.VMEM((K, D), jnp.float32),
        pltpu.VMEM((K, D), jnp.float32),
        pltpu.VMEM((K, D), jnp.float32),
        pltpu.VMEM((K, D), jnp.float32),
        pltpu.VMEM((K, D), jnp.float32),
        pltpu.SemaphoreType.DMA,
        pltpu.SemaphoreType.DMA,
        pltpu.SemaphoreType.DMA,
        pltpu.SemaphoreType.DMA,
        pltpu.SemaphoreType.DMA,
        pltpu.SemaphoreType.DMA,
    ],
)


def _l1_body(dego_ref, degi_ref, x_ref, w_ref, z0_ref, z1_ref,
             ns_ref, nd_ref):
    nsrc = lax.rsqrt(jnp.maximum(dego_ref[..., 0:1], 1.0))
    ndst = lax.rsqrt(jnp.maximum(degi_ref[..., 0:1], 1.0))
    ns_ref[...] = nsrc
    nd_ref[...] = ndst
    z = jnp.dot(x_ref[...] * nsrc, w_ref[...],
                preferred_element_type=jnp.float32)
    z0_ref[...] = z[:, :D]
    z1_ref[...] = z[:, D:]


_l1 = pl.pallas_call(
    _l1_body,
    grid=(GRID,),
    in_specs=[
        pl.BlockSpec((BN, D), lambda i: (i, 0)),
        pl.BlockSpec((BN, D), lambda i: (i, 0)),
        pl.BlockSpec((BN, IN), lambda i: (i, 0)),
        pl.BlockSpec((IN, H), lambda i: (0, 0)),
    ],
    out_specs=[
        pl.BlockSpec((BN, D), lambda i: (i, 0)),
        pl.BlockSpec((BN, D), lambda i: (i, 0)),
        pl.BlockSpec((BN, 1), lambda i: (i, 0)),
        pl.BlockSpec((BN, 1), lambda i: (i, 0)),
    ],
    out_shape=[
        _SDS((NROW, D), jnp.float32),
        _SDS((NROW, D), jnp.float32),
        _SDS((N, 1), jnp.float32),
        _SDS((N, 1), jnp.float32),
    ],
)


def _make_mid(hout):
    dh = hout // 2
    wpad = D - dh

    def body(a0_ref, a1_ref, nd_ref, ns_ref, b_ref, w_ref, z0_ref, z1_ref):
        h = jnp.concatenate([a0_ref[...], a1_ref[...]], axis=1)
        h = jnp.maximum(h * nd_ref[...] + b_ref[...], 0.0)
        z = jnp.dot(h * ns_ref[...], w_ref[...],
                    preferred_element_type=jnp.float32)
        if wpad:
            zp = jnp.zeros((z.shape[0], wpad), jnp.float32)
            z0_ref[...] = jnp.concatenate([z[:, :dh], zp], axis=1)
            z1_ref[...] = jnp.concatenate([z[:, dh:], zp], axis=1)
        else:
            z0_ref[...] = z[:, :dh]
            z1_ref[...] = z[:, dh:]

    return pl.pallas_call(
        body,
        grid=(GRID,),
        in_specs=[
            pl.BlockSpec((BN, D), lambda i: (i, 0)),
            pl.BlockSpec((BN, D), lambda i: (i, 0)),
            pl.BlockSpec((BN, 1), lambda i: (i, 0)),
            pl.BlockSpec((BN, 1), lambda i: (i, 0)),
            pl.BlockSpec((1, H), lambda i: (0, 0)),
            pl.BlockSpec((H, hout), lambda i: (0, 0)),
        ],
        out_specs=[
            pl.BlockSpec((BN, D), lambda i: (i, 0)),
            pl.BlockSpec((BN, D), lambda i: (i, 0)),
        ],
        out_shape=[
            _SDS((NROW, D), jnp.float32),
            _SDS((NROW, D), jnp.float32),
        ],
    )


_mid256 = _make_mid(H)
_mid128 = _make_mid(OUT)


def _fin_body(a0_ref, a1_ref, nd_ref, b_ref, o_ref):
    i = pl.program_id(0)
    h = jnp.concatenate([a0_ref[..., :OUT // 2], a1_ref[..., :OUT // 2]],
                        axis=1) * nd_ref[...]
    p = jnp.sum(h, axis=0, keepdims=True)

    @pl.when(i == 0)
    def _():
        o_ref[...] = jnp.zeros_like(o_ref)

    o_ref[...] += p

    @pl.when(i == GRID - 1)
    def _():
        o_ref[...] = o_ref[...] * (1.0 / N) + b_ref[...]


_fin = pl.pallas_call(
    _fin_body,
    grid=(GRID,),
    in_specs=[
        pl.BlockSpec((BN, D), lambda i: (i, 0)),
        pl.BlockSpec((BN, D), lambda i: (i, 0)),
        pl.BlockSpec((BN, 1), lambda i: (i, 0)),
        pl.BlockSpec((1, OUT), lambda i: (0, 0)),
    ],
    out_specs=pl.BlockSpec((1, OUT), lambda i: (0, 0)),
    out_shape=_SDS((1, OUT), jnp.float32),
)


def kernel(x, edge_index, W1, b1, W2, b2, W3, b3, W4, b4):
    src = edge_index[0]
    dst = edge_index[1]
    npad = EPAD - E
    fill = jnp.int32(N) + (jnp.arange(npad, dtype=jnp.int32) % 16)
    srcp = jnp.concatenate([src, fill]).reshape(NS, NB, K)
    dstp = jnp.concatenate([dst, fill]).reshape(NS, NB, K)
    lslab = jnp.stack([srcp, dstp], axis=2)
    lslab = jnp.stack([lslab, lslab])
    dslab = jnp.stack([jnp.stack([srcp, srcp], axis=2),
                       jnp.stack([dstp, dstp], axis=2)])

    ones = jnp.ones((NROW, D), jnp.float32)
    dego, degi = _agg(ones, ones, dslab)
    z0, z1, nsrc, ndst = _l1(dego, degi, x, W1)
    a0, a1 = _agg(z0, z1, lslab)
    z0, z1 = _mid256(a0, a1, ndst, nsrc, b1.reshape(1, H), W2)
    a0, a1 = _agg(z0, z1, lslab)
    z0, z1 = _mid256(a0, a1, ndst, nsrc, b2.reshape(1, H), W3)
    a0, a1 = _agg(z0, z1, lslab)
    y0, y1 = _mid128(a0, a1, ndst, nsrc, b3.reshape(1, H), W4)
    a0, a1 = _agg(y0, y1, lslab)
    out = _fin(a0, a1, ndst, b4.reshape(1, OUT))
    return out.reshape(OUT)
